# baseline (speedup 1.0000x reference)
import functools

import jax
import jax.numpy as jnp
from jax import lax
from jax.experimental import pallas as pl
from jax.experimental.pallas import tpu as pltpu

_W1 = 1.0
_W2 = 1.0
_S1 = 0.5
_S2 = 0.5

_B = 16384
_C = 1000
_BR = 512
_NBLK = _B // _BR
_NCC = 8


def _tc_body(pred_ref, tgt_ref, marg_ref, out_ref):
    j = pl.program_id(1)
    t = tgt_ref[...]
    m = marg_ref[...]
    lt = t - j * 128
    li = jax.lax.broadcasted_iota(jnp.int32, (_BR, 128), 1)
    n = jnp.sum(jnp.where(li == lt, pred_ref[...], 0.0),
                axis=1, keepdims=True)
    w1 = _W1 * jnp.exp(-_S1 * m * m)
    w2 = _W2 * jnp.exp(-_S2 * m * m)
    contrib = jnp.where(m > 0, w1 * n, 0.0) + jnp.where(m < 0, w2 * n, 0.0)

    @pl.when(j == 0)
    def _():
        out_ref[...] = jnp.zeros((1, 1, 128), jnp.float32)

    out_ref[...] += jnp.broadcast_to(jnp.sum(contrib), (1, 1, 128))


_tc_partials = pl.pallas_call(
    _tc_body,
    out_shape=jax.ShapeDtypeStruct((_NBLK, 1, 128), jnp.float32),
    grid=(_NBLK, _NCC),
    in_specs=[
        pl.BlockSpec((_BR, 128), lambda i, j: (i, j)),
        pl.BlockSpec((_BR, 1), lambda i, j: (i, 0)),
        pl.BlockSpec((_BR, 1), lambda i, j: (i, 0)),
    ],
    out_specs=pl.BlockSpec((1, 1, 128), lambda i, j: (i, 0, 0)),
)


def kernel(preds, targets, margin):
    partials = _tc_partials(preds, targets[:, None], margin[:, None])
    return -jnp.sum(partials[:, 0, 0]) / margin.shape[0]


# speedup vs baseline: 2.7627x; 2.7627x over previous
import functools

import jax
import jax.numpy as jnp
from jax import lax
from jax.experimental import pallas as pl
from jax.experimental.pallas import tpu as pltpu
from jax.experimental.pallas import tpu_sc as plsc

_W1 = 1.0
_W2 = 1.0
_S1 = 0.5
_S2 = 0.5

_B = 16384
_C = 1000
_NC, _NS, _L = 2, 16, 16
_NW = _NC * _NS
_BPW = _B // _NW
_NG = 8
_K = 64
_NR = _BPW // _K


def _sc_body(preds_hbm, aux_hbm, tgt_hbm, marg_hbm, out_hbm,
             lists_v, vals_v, tgt_v, marg_v, rank_v, cnt_v, acc_v, sem):
    wid = lax.axis_index("s") * _NC + lax.axis_index("c")
    base = wid * _BPW

    pltpu.sync_copy(tgt_hbm.at[pl.ds(base, _BPW)], tgt_v)
    pltpu.sync_copy(marg_hbm.at[pl.ds(base, _BPW)], marg_v)

    lane = lax.iota(jnp.int32, _L)
    zero_i = jnp.zeros((_L,), jnp.int32)
    cnt_v[...] = zero_i

    for g in range(_NG):
        @plsc.parallel_loop(0, _NR * (_K // _L))
        def _memset(j, g=g):
            r = jax.lax.shift_right_logical(j, 2)
            l = jax.lax.bitwise_and(j, jnp.int32(_K // _L - 1))
            off = j * _L
            lists_v[g, r, pl.ds(pl.multiple_of(l * _L, _L), _L)] = (
                base + off + lane)

    def _rank(k, carry):
        off = k * _L
        t = tgt_v[pl.ds(pl.multiple_of(off, _L), _L)]
        gv = jax.lax.shift_right_logical(t, 7)
        rank_in = jnp.zeros((_L,), jnp.int32)
        total = jnp.zeros((_L,), jnp.int32)
        for l in range(_L):
            g_l = jax.lax.squeeze(jax.lax.slice(gv, (l,), (l + 1,)), (0,))
            same = (gv == g_l).astype(jnp.int32)
            rank_in = rank_in + jnp.where(lane > l, same, 0)
            total = total + same
        pre = plsc.load_gather(cnt_v, [gv])
        rank = pre + rank_in
        plsc.store_scatter(
            lists_v,
            [gv, jax.lax.shift_right_logical(rank, 6),
             jax.lax.bitwise_and(rank, jnp.int32(_K - 1))],
            base + off + lane)
        rank_v[pl.ds(pl.multiple_of(off, _L), _L)] = rank
        is_last = rank_in + 1 == total
        plsc.addupdate_scatter(cnt_v, [gv], total, mask=is_last)
        return carry

    lax.fori_loop(0, _BPW // _L, _rank, 0)

    counts = cnt_v[...]
    max_count = jax.lax.reduce_max(counts, axes=(0,))

    def _desc(g, r):
        return pltpu.make_async_copy(
            preds_hbm.at[lists_v.at[g, r],
                         pl.ds(pl.multiple_of(g * 128, 128), 128)],
            vals_v.at[g],
            sem,
        )

    def _tail_desc(r):
        return pltpu.make_async_copy(
            aux_hbm.at[lists_v.at[_NG - 1, r], pl.ds(0, 128)],
            vals_v.at[_NG - 1],
            sem,
        )

    def _round(carry):
        r, acc = carry

        def _fire(g, c):
            _desc(g, r).start()
            return c

        lax.fori_loop(0, _NG - 1, _fire, 0)
        _tail_desc(r).start()

        def _drain(g, c):
            _desc(g, r).wait()
            return c

        lax.fori_loop(0, _NG - 1, _drain, 0)
        _tail_desc(r).wait()

        def _pick(k, acc_in):
            off = k * _L
            t = tgt_v[pl.ds(pl.multiple_of(off, _L), _L)]
            m = marg_v[pl.ds(pl.multiple_of(off, _L), _L)]
            rank = rank_v[pl.ds(pl.multiple_of(off, _L), _L)]
            gv = jax.lax.shift_right_logical(t, 7)
            col = jnp.where(t >= 896, t - 896,
                            jax.lax.bitwise_and(t, jnp.int32(127)))
            rw = rank - r * _K
            win = jnp.logical_and(rw >= 0, rw < _K)
            rw = jnp.clip(rw, 0, _K - 1)
            v = plsc.load_gather(vals_v, [gv, rw, col], mask=win)
            pos = jnp.logical_and(win, m > 0)
            neg = jnp.logical_and(win, m < 0)
            w1 = _W1 * jnp.exp(-_S1 * m * m)
            w2 = _W2 * jnp.exp(-_S2 * m * m)
            return (acc_in + jnp.where(pos, w1 * v, 0.0)
                    + jnp.where(neg, w2 * v, 0.0))

        acc = lax.fori_loop(0, _BPW // _L, _pick, acc)
        return r + 1, acc

    def _cond(carry):
        r, _ = carry
        return r * _K < max_count

    _, acc = lax.while_loop(_cond, _round,
                            (jnp.int32(0), jnp.zeros((_L,), jnp.float32)))
    acc_v[...] = acc
    pltpu.sync_copy(acc_v, out_hbm.at[wid])


@functools.partial(
    pl.kernel,
    out_type=jax.ShapeDtypeStruct((_NW, _L), jnp.float32),
    mesh=plsc.VectorSubcoreMesh(core_axis_name="c", subcore_axis_name="s"),
    compiler_params=pltpu.CompilerParams(needs_layout_passes=False, skip_device_barrier=True),
    scratch_types=[
        pltpu.VMEM((_NG, _NR, _K), jnp.int32),
        pltpu.VMEM((_NG, _K, 128), jnp.float32),
        pltpu.VMEM((_BPW,), jnp.int32),
        pltpu.VMEM((_BPW,), jnp.float32),
        pltpu.VMEM((_BPW,), jnp.int32),
        pltpu.VMEM((_L,), jnp.int32),
        pltpu.VMEM((_L,), jnp.float32),
        pltpu.SemaphoreType.DMA,
    ],
)
def _sc_partial_sums(preds_hbm, aux_hbm, tgt_hbm, marg_hbm, out_hbm,
                     lists_v, vals_v, tgt_v, marg_v, rank_v, cnt_v, acc_v,
                     sem):
    _sc_body(preds_hbm, aux_hbm, tgt_hbm, marg_hbm, out_hbm,
             lists_v, vals_v, tgt_v, marg_v, rank_v, cnt_v, acc_v, sem)


def kernel(preds, targets, margin):
    aux = jnp.pad(preds[:, 896:], ((0, 0), (0, 1024 - _C)))
    partials = _sc_partial_sums(preds, aux, targets, margin)
    return -jnp.sum(partials) / margin.shape[0]
